# baseline (speedup 1.0000x reference)
import dataclasses
import functools

import jax
import jax.numpy as jnp
from jax import lax
from jax.experimental import pallas as pl
from jax.experimental.pallas import tpu as pltpu
from jax.experimental.pallas import tpu_sc as plsc

VOCAB = 30522
D = 64
B = 16384
L = 200

NC = 2
NS = 16
LANES = 16
NW = NC * NS
ROWS_W = B // NW
R_STAGE = 128
NSTAGE = ROWS_W // R_STAGE
NB = 8
GL1 = 128
GL2 = L - GL1
INV_L = 1.0 / L


def _compiler_params():
    cp = pltpu.CompilerParams()
    for field, val in (("needs_layout_passes", False),
                       ("use_tc_tiling_on_sc", False)):
        if field in pltpu.CompilerParams.__dataclass_fields__:
            cp = dataclasses.replace(cp, **{field: val})
    return cp


def _sc_embed_mean(ids, table):
    mesh = plsc.VectorSubcoreMesh(core_axis_name="c", subcore_axis_name="s")

    @functools.partial(
        pl.kernel,
        mesh=mesh,
        compiler_params=_compiler_params(),
        out_type=jax.ShapeDtypeStruct((B * D,), jnp.float32),
        scratch_types=[
            pltpu.VMEM((2, R_STAGE * L), jnp.int32),
            pltpu.VMEM((NB, L, D), jnp.bfloat16),
            pltpu.VMEM((R_STAGE * D,), jnp.float32),
        ] + [pltpu.SemaphoreType.DMA] * NB,
    )
    def k(ids_hbm, table_hbm, out_hbm, ids_v, rbuf, obuf, *gsem):
        c = lax.axis_index("c")
        s = lax.axis_index("s")
        row0_g = (c * NS + s) * ROWS_W

        def load_stage(st):
            off = pl.multiple_of((row0_g + st * R_STAGE) * L, 8)
            pltpu.sync_copy(
                ids_hbm.at[pl.ds(off, R_STAGE * L)],
                ids_v.at[st % 2],
            )

        def start_gathers(b, idh, rl):
            idrow = ids_v.at[idh]
            pltpu.async_copy(
                table_hbm.at[idrow.at[pl.ds(rl * L, GL1)]],
                rbuf.at[b, pl.ds(0, GL1)], gsem[b],
            )
            pltpu.async_copy(
                table_hbm.at[idrow.at[pl.ds(rl * L + GL1, GL2)]],
                rbuf.at[b, pl.ds(GL1, GL2)], gsem[b],
            )

        def wait_gathers(b, idh, rl):
            idrow = ids_v.at[idh]
            pltpu.make_async_copy(
                table_hbm.at[idrow.at[pl.ds(rl * L, GL1)]],
                rbuf.at[b, pl.ds(0, GL1)], gsem[b],
            ).wait()
            pltpu.make_async_copy(
                table_hbm.at[idrow.at[pl.ds(rl * L + GL1, GL2)]],
                rbuf.at[b, pl.ds(GL1, GL2)], gsem[b],
            ).wait()

        zero = jnp.zeros((LANES,), jnp.float32)

        def process(st, rl, b, lookahead):
            wait_gathers(b, st % 2, rl)

            def acc_body(t, acc):
                x0 = rbuf[b, t, pl.ds(0, 2 * LANES)]
                x1 = rbuf[b, t, pl.ds(2 * LANES, 2 * LANES)]
                y0 = rbuf[b, t + 1, pl.ds(0, 2 * LANES)]
                y1 = rbuf[b, t + 1, pl.ds(2 * LANES, 2 * LANES)]
                s0 = x0 + y0
                s1 = x1 + y1
                a0, a1 = plsc.unpack(s0, format=plsc.PackFormat.INTERLEAVED)
                a2, a3 = plsc.unpack(s1, format=plsc.PackFormat.INTERLEAVED)
                return (acc[0] + a0, acc[1] + a1, acc[2] + a2, acc[3] + a3)

            accs = plsc.parallel_loop(
                0, L, step=2, unroll=4, carry=(zero, zero, zero, zero)
            )(acc_body)

            for q in range(4):
                obuf[pl.ds(rl * D + q * LANES, LANES)] = accs[q] * INV_L

            if lookahead is not None:
                nidh, nrl = lookahead
                start_gathers(b, nidh, nrl)

        load_stage(0)
        if NSTAGE > 1:
            load_stage(1)
        for b in range(NB):
            start_gathers(b, 0, b)

        for st in range(NSTAGE):
            if 1 <= st and st + 1 < NSTAGE:
                load_stage(st + 1)

            @pl.loop(0, R_STAGE - NB, step=NB)
            def _(rl0):
                for bb in range(NB):
                    process(st, rl0 + bb, bb, (st % 2, rl0 + bb + NB))

            for bb in range(NB):
                rl = R_STAGE - NB + bb
                la = ((st + 1) % 2, bb) if st + 1 < NSTAGE else None
                process(st, rl, bb, la)

            pltpu.sync_copy(
                obuf,
                out_hbm.at[pl.ds(
                    pl.multiple_of((row0_g + st * R_STAGE) * D, 8),
                    R_STAGE * D,
                )],
            )

    return k(ids, table)


def kernel(input_ids, table):
    table_pre = (
        table.reshape(VOCAB, 2, 2, LANES)
        .transpose(0, 1, 3, 2)
        .reshape(VOCAB, D)
        .astype(jnp.bfloat16)
    )
    ids1d = input_ids.astype(jnp.int32).reshape(B * L)
    return _sc_embed_mean(ids1d, table_pre).reshape(B, D)
